# Initial kernel scaffold; baseline (speedup 1.0000x reference)
import jax
import jax.numpy as jnp
from jax.experimental import pallas as pl


def kernel(logits, gt, xyz, T1, T2, k, b):
    raise NotImplementedError("write your pallas kernel here")


# speedup vs baseline: 1.0524x; 1.0524x over previous
import jax
import jax.numpy as jnp
from jax import lax
from jax.experimental import pallas as pl
from jax.experimental.pallas import tpu as pltpu
from jax.experimental.pallas import tpu_sc as plsc

_THRESHOLD = 6.43

_R = 256

_NC = 2
_NS = 16
_NW = _NC * _NS
_K = 64


def _entropy_scale_body(t1_ref, t2_ref, k_ref, b_ref, x_ref, xyz_ref,
                        cal_ref, cond_ref):
    x = x_ref[...]
    m = jnp.max(x, axis=1, keepdims=True)
    e = jnp.exp(x - m)
    s = jnp.sum(e, axis=1, keepdims=True)
    p = e / s
    score = jnp.sum(-p * jnp.log(p), axis=1)
    cond = score < _THRESHOLD
    xyz = xyz_ref[...]
    depth = jnp.sqrt(jnp.sum(jnp.abs(xyz) ** 2, axis=1))
    coeff = k_ref[0, 0] * depth + b_ref[0, 0]
    temp = jnp.where(cond, t1_ref[0, 0] * coeff, t2_ref[0, 0] * coeff)
    cal_ref[...] = x / temp[:, None]
    cond_ref[...] = cond.astype(jnp.float32).reshape(1, 1, _R)


def _dest_body(cond_ref, dest_ref):
    cnd = cond_ref[...]
    ssz = cnd.shape[0]
    ir = lax.broadcasted_iota(jnp.int32, (ssz, ssz), 0).astype(jnp.float32)
    ic = lax.broadcasted_iota(jnp.int32, (ssz, ssz), 1).astype(jnp.float32)
    upper = (ir < ic).astype(jnp.float32)
    lower = (ir > ic).astype(jnp.float32)
    row_tot = jnp.sum(cnd, axis=1, keepdims=True)
    offs = jnp.dot(lower, row_tot, preferred_element_type=jnp.float32)
    inrow = jnp.dot(cnd, upper, preferred_element_type=jnp.float32)
    pc = inrow + offs
    total = jnp.sum(cnd)
    g = ir * jnp.float32(ssz) + ic
    dest = jnp.where(cnd > 0.5, pc, total + (g - pc))
    dest_ref[...] = dest.astype(jnp.int32)


def _scatter_body(cal_hbm, dest_hbm, gt_hbm, out_hbm, outgt_hbm,
                  rows_v, idx_v, gt_v, sem):
    n = cal_hbm.shape[0]
    rows_per_w = n // _NW
    nchunk = rows_per_w // _K
    wid = lax.axis_index("s") * _NC + lax.axis_index("c")

    def body(ci, carry):
        base = wid * rows_per_w + ci * _K
        pltpu.sync_copy(cal_hbm.at[pl.ds(base, _K)], rows_v)
        pltpu.sync_copy(dest_hbm.at[pl.ds(base, _K)], idx_v)
        pltpu.sync_copy(gt_hbm.at[pl.ds(base, _K)], gt_v)
        pltpu.async_copy(rows_v, out_hbm.at[idx_v], sem).wait()
        pltpu.async_copy(gt_v, outgt_hbm.at[idx_v], sem).wait()
        return carry

    lax.fori_loop(0, nchunk, body, 0)


def kernel(logits, gt, xyz, T1, T2, k, b):
    n, c = logits.shape
    grid = n // _R
    smem_spec = pl.BlockSpec((1, 1), lambda i: (0, 0), memory_space=pltpu.SMEM)
    cal, condf = pl.pallas_call(
        _entropy_scale_body,
        grid=(grid,),
        in_specs=[
            smem_spec, smem_spec, smem_spec, smem_spec,
            pl.BlockSpec((_R, c), lambda i: (i, 0)),
            pl.BlockSpec((_R, 3), lambda i: (i, 0)),
        ],
        out_specs=[
            pl.BlockSpec((_R, c), lambda i: (i, 0)),
            pl.BlockSpec((1, 1, _R), lambda i: (i, 0, 0)),
        ],
        out_shape=[
            jax.ShapeDtypeStruct((n, c), jnp.float32),
            jax.ShapeDtypeStruct((grid, 1, _R), jnp.float32),
        ],
    )(T1.reshape(1, 1), T2.reshape(1, 1), k.reshape(1, 1), b.reshape(1, 1),
      logits, xyz)

    ssz = 256
    assert ssz * ssz == n
    dest = pl.pallas_call(
        _dest_body,
        out_shape=jax.ShapeDtypeStruct((ssz, ssz), jnp.int32),
    )(condf.reshape(ssz, ssz))
    dest = dest.reshape(n)

    scatter = pl.kernel(
        _scatter_body,
        out_type=(
            jax.ShapeDtypeStruct((n, c), jnp.float32),
            jax.ShapeDtypeStruct((n,), jnp.int32),
        ),
        mesh=plsc.VectorSubcoreMesh(core_axis_name="c", subcore_axis_name="s"),
        scratch_types=[
            pltpu.VMEM((_K, c), jnp.float32),
            pltpu.VMEM((_K,), jnp.int32),
            pltpu.VMEM((_K,), jnp.int32),
            pltpu.SemaphoreType.DMA,
        ],
    )
    out_cal, out_gt = scatter(cal, dest, gt)
    return (out_cal, out_gt)


# speedup vs baseline: 1.1365x; 1.0800x over previous
import jax
import jax.numpy as jnp
from jax import lax
from jax.experimental import pallas as pl
from jax.experimental.pallas import tpu as pltpu
from jax.experimental.pallas import tpu_sc as plsc

_THRESHOLD = 6.43

_R = 256

_NC = 2
_NS = 16
_NW = _NC * _NS
_K = 32
_NB = 2


def _entropy_temp_body(t1_ref, t2_ref, k_ref, b_ref, x_ref, xyz_ref,
                       trep_ref, cond_ref):
    x = x_ref[...]
    m = jnp.max(x, axis=1, keepdims=True)
    e = jnp.exp(x - m)
    s = jnp.sum(e, axis=1, keepdims=True)
    p = e / s
    score = jnp.sum(-p * jnp.log(p), axis=1)
    cond = score < _THRESHOLD
    xyz = xyz_ref[...]
    depth = jnp.sqrt(jnp.sum(jnp.abs(xyz) ** 2, axis=1))
    coeff = k_ref[0, 0] * depth + b_ref[0, 0]
    temp = jnp.where(cond, t1_ref[0, 0] * coeff, t2_ref[0, 0] * coeff)
    trep_ref[...] = jnp.broadcast_to(temp[:, None], (_R, 16))
    cond_ref[...] = cond.astype(jnp.float32).reshape(1, 1, _R)


def _dest_body(cond_ref, dest_ref):
    cnd = cond_ref[...]
    ssz = cnd.shape[0]
    ir = lax.broadcasted_iota(jnp.int32, (ssz, ssz), 0).astype(jnp.float32)
    ic = lax.broadcasted_iota(jnp.int32, (ssz, ssz), 1).astype(jnp.float32)
    upper = (ir < ic).astype(jnp.float32)
    lower = (ir > ic).astype(jnp.float32)
    row_tot = jnp.sum(cnd, axis=1, keepdims=True)
    offs = jnp.dot(lower, row_tot, preferred_element_type=jnp.float32)
    inrow = jnp.dot(cnd, upper, preferred_element_type=jnp.float32)
    pc = inrow + offs
    total = jnp.sum(cnd)
    g = ir * jnp.float32(ssz) + ic
    dest = jnp.where(cnd > 0.5, pc, total + (g - pc))
    dest_ref[...] = dest.astype(jnp.int32)


def _scale_scatter_body(x_hbm, trep_hbm, dest_hbm, gt_hbm, out_hbm, outgt_hbm,
                        rows_v, trep_v, idx_v, gt_v,
                        load_sem, row_sem, gt_sem):
    n = x_hbm.shape[0]
    c = x_hbm.shape[1]
    rows_per_w = n // _NW
    nchunk = rows_per_w // _K
    wid = lax.axis_index("s") * _NC + lax.axis_index("c")
    w0 = wid * rows_per_w

    def start_load(ci, bf):
        base = w0 + ci * _K
        pltpu.async_copy(x_hbm.at[pl.ds(base, _K)], rows_v.at[bf], load_sem[bf])
        pltpu.async_copy(trep_hbm.at[pl.ds(base, _K)], trep_v.at[bf], load_sem[bf])
        pltpu.async_copy(dest_hbm.at[pl.ds(base, _K)], idx_v.at[bf], load_sem[bf])
        pltpu.async_copy(gt_hbm.at[pl.ds(base, _K)], gt_v.at[bf], load_sem[bf])

    def wait_load(ci, bf):
        base = w0 + ci * _K
        pltpu.make_async_copy(x_hbm.at[pl.ds(base, _K)], rows_v.at[bf], load_sem[bf]).wait()
        pltpu.make_async_copy(trep_hbm.at[pl.ds(base, _K)], trep_v.at[bf], load_sem[bf]).wait()
        pltpu.make_async_copy(dest_hbm.at[pl.ds(base, _K)], idx_v.at[bf], load_sem[bf]).wait()
        pltpu.make_async_copy(gt_hbm.at[pl.ds(base, _K)], gt_v.at[bf], load_sem[bf]).wait()

    def divide(bf):
        def rbody(r, carry):
            tv = trep_v[bf, r]
            for cc in range(c // 16):
                sl = (bf, r, pl.ds(cc * 16, 16))
                rows_v[sl] = rows_v[sl] / tv
            return carry
        lax.fori_loop(0, _K, rbody, 0)

    def fire_scatter(bf):
        pltpu.async_copy(rows_v.at[bf], out_hbm.at[idx_v.at[bf]], row_sem[bf])
        pltpu.async_copy(gt_v.at[bf], outgt_hbm.at[idx_v.at[bf]], gt_sem[bf])

    def wait_scatter(bf):
        pltpu.make_async_copy(rows_v.at[bf], out_hbm.at[idx_v.at[bf]], row_sem[bf]).wait()
        pltpu.make_async_copy(gt_v.at[bf], outgt_hbm.at[idx_v.at[bf]], gt_sem[bf]).wait()

    start_load(0, 0)

    def outer(ci2, carry):
        ci = ci2 * _NB
        for b in range(_NB):
            cib = ci + b
            nb = (b + 1) % _NB
            wait_load(cib, b)
            divide(b)
            fire_scatter(b)
            @pl.when(cib + 1 < nchunk)
            def _():
                @pl.when(cib >= 1)
                def _():
                    wait_scatter(nb)
                start_load(cib + 1, nb)
        return carry

    lax.fori_loop(0, nchunk // _NB, outer, 0)
    wait_scatter(0)
    wait_scatter(1)


def kernel(logits, gt, xyz, T1, T2, k, b):
    n, c = logits.shape
    grid = n // _R
    smem_spec = pl.BlockSpec((1, 1), lambda i: (0, 0), memory_space=pltpu.SMEM)
    trep, condf = pl.pallas_call(
        _entropy_temp_body,
        grid=(grid,),
        in_specs=[
            smem_spec, smem_spec, smem_spec, smem_spec,
            pl.BlockSpec((_R, c), lambda i: (i, 0)),
            pl.BlockSpec((_R, 3), lambda i: (i, 0)),
        ],
        out_specs=[
            pl.BlockSpec((_R, 16), lambda i: (i, 0)),
            pl.BlockSpec((1, 1, _R), lambda i: (i, 0, 0)),
        ],
        out_shape=[
            jax.ShapeDtypeStruct((n, 16), jnp.float32),
            jax.ShapeDtypeStruct((grid, 1, _R), jnp.float32),
        ],
    )(T1.reshape(1, 1), T2.reshape(1, 1), k.reshape(1, 1), b.reshape(1, 1),
      logits, xyz)

    ssz = 256
    assert ssz * ssz == n
    dest = pl.pallas_call(
        _dest_body,
        out_shape=jax.ShapeDtypeStruct((ssz, ssz), jnp.int32),
    )(condf.reshape(ssz, ssz))
    dest = dest.reshape(n)

    scatter = pl.kernel(
        _scale_scatter_body,
        out_type=(
            jax.ShapeDtypeStruct((n, c), jnp.float32),
            jax.ShapeDtypeStruct((n,), jnp.int32),
        ),
        mesh=plsc.VectorSubcoreMesh(core_axis_name="c", subcore_axis_name="s"),
        scratch_types=[
            pltpu.VMEM((_NB, _K, c), jnp.float32),
            pltpu.VMEM((_NB, _K, 16), jnp.float32),
            pltpu.VMEM((_NB, _K), jnp.int32),
            pltpu.VMEM((_NB, _K), jnp.int32),
            [pltpu.SemaphoreType.DMA] * _NB,
            [pltpu.SemaphoreType.DMA] * _NB,
            [pltpu.SemaphoreType.DMA] * _NB,
        ],
    )
    out_cal, out_gt = scatter(logits, trep, dest, gt)
    return (out_cal, out_gt)
